# baseline (speedup 1.0000x reference)
import functools

import jax
import jax.numpy as jnp
from jax import lax
from jax.experimental import pallas as pl
from jax.experimental.pallas import tpu as pltpu
from jax.experimental.pallas import tpu_sc as plsc

BATCH = 4096
IN_DIM = 128
INTER_DIM = 128
VOCAB = 1000
CH = 8
CPER = 16
CMAX = 1536
BP = BATCH + CH
RM = 512


def _chunk_matvec_body(widx_ref, rstart_ref, *refs):
    dw_refs = refs[:CPER]
    c_ref, out_ref = refs[CPER:]
    i = pl.program_id(0)
    for j in range(CPER):
        k = rstart_ref[i * CPER + j]

        @pl.when(k < BATCH)
        def _(j=j, k=k):
            w = dw_refs[j][0].astype(jnp.bfloat16)
            rows = c_ref[pl.ds(k, CH), :].astype(jnp.bfloat16)
            out_ref[pl.ds(k, CH), :] = jnp.tanh(
                jax.lax.dot(rows, w, preferred_element_type=jnp.float32))


L = 16
VP = 1024


def _sc_sort_body(ids_hbm, perm_hbm, inv_hbm, widx_hbm, rstart_hbm,
                  ids_v, hist_v, perm_v, inv_v, starts_v, ncum_v,
                  wi_v, rs_v):
    wid = lax.axis_index("s") * 2 + lax.axis_index("c")

    @pl.when(wid == 0)
    def _():
        pltpu.sync_copy(ids_hbm, ids_v)
        iota = lax.iota(jnp.int32, L)
        zeros = jnp.zeros((L,), jnp.int32)
        ones = jnp.ones((L,), jnp.int32)

        def zero_body(k, c):
            hist_v[pl.ds(k * L, L)] = zeros
            return c

        lax.fori_loop(0, (L * VP) // L, zero_body, 0)

        def hist_body(c, carry):
            vec = ids_v[pl.ds(c * L, L)]
            plsc.addupdate_scatter(hist_v, [vec + iota * VP], ones)
            return carry

        lax.fori_loop(0, BATCH // L, hist_body, 0)

        def mark_init(c, carry):
            wi_v[pl.ds(c * L, L)] = zeros - 1
            return carry

        lax.fori_loop(0, CMAX // L, mark_init, 0)

        def merge_body(k, carry):
            rcarry, ccarry = carry
            acc = hist_v[pl.ds(k * L, L)]
            for l in range(1, L):
                acc = acc + hist_v[pl.ds(l * VP + k * L, L)]
            incl = plsc.cumsum(acc)
            svec = incl - acc + rcarry
            starts_v[pl.ds(k * L, L)] = svec
            base = svec
            for l in range(L):
                tmp = hist_v[pl.ds(l * VP + k * L, L)]
                hist_v[pl.ds(l * VP + k * L, L)] = base
                base = base + tmp
            nc = lax.shift_right_logical(acc + (CH - 1), 3)
            ninc = plsc.cumsum(nc)
            nprev = ninc - nc + ccarry
            ncum_v[pl.ds(k * L, L)] = nprev
            plsc.store_scatter(wi_v, [nprev], k * L + iota, mask=acc > 0)
            return (rcarry + jnp.sum(acc), ccarry + jnp.sum(nc))

        lax.fori_loop(0, VP // L, merge_body, (0, 0))

        def fill_body(c, carry):
            vec = wi_v[pl.ds(c * L, L)]
            g = jnp.maximum(plsc.cummax(vec), carry)
            nprev = plsc.load_gather(ncum_v, [g])
            sg = plsc.load_gather(starts_v, [g])
            slot = c * L + iota
            rs = jnp.minimum(sg + CH * (slot - nprev), BATCH)
            wi_v[pl.ds(c * L, L)] = g
            rs_v[pl.ds(c * L, L)] = rs
            return jnp.max(g)

        lax.fori_loop(0, CMAX // L, fill_body, 0)

        def place_body(c, carry):
            vec = ids_v[pl.ds(c * L, L)]
            idx = vec + iota * VP
            cur = plsc.load_gather(hist_v, [idx])
            rows = c * L + iota
            plsc.store_scatter(perm_v, [cur], rows)
            inv_v[pl.ds(c * L, L)] = cur
            plsc.store_scatter(hist_v, [idx], cur + 1)
            return carry

        lax.fori_loop(0, BATCH // L, place_body, 0)

        pltpu.sync_copy(perm_v, perm_hbm)
        pltpu.sync_copy(inv_v, inv_hbm)
        pltpu.sync_copy(wi_v, widx_hbm)
        pltpu.sync_copy(rs_v, rstart_hbm)


@functools.partial(
    pl.kernel,
    out_type=[
        jax.ShapeDtypeStruct((BATCH,), jnp.int32),
        jax.ShapeDtypeStruct((BATCH,), jnp.int32),
        jax.ShapeDtypeStruct((CMAX,), jnp.int32),
        jax.ShapeDtypeStruct((CMAX,), jnp.int32),
    ],
    mesh=plsc.VectorSubcoreMesh(core_axis_name="c", subcore_axis_name="s"),
    compiler_params=pltpu.CompilerParams(needs_layout_passes=False),
    scratch_types=[
        pltpu.VMEM((BATCH,), jnp.int32),
        pltpu.VMEM((L * VP,), jnp.int32),
        pltpu.VMEM((BATCH,), jnp.int32),
        pltpu.VMEM((BATCH,), jnp.int32),
        pltpu.VMEM((VP,), jnp.int32),
        pltpu.VMEM((VP,), jnp.int32),
        pltpu.VMEM((CMAX,), jnp.int32),
        pltpu.VMEM((CMAX,), jnp.int32),
    ],
)
def _sc_sort(ids_hbm, perm_hbm, inv_hbm, widx_hbm, rstart_hbm,
             ids_v, hist_v, perm_v, inv_v, starts_v, ncum_v, wi_v, rs_v):
    _sc_sort_body(ids_hbm, perm_hbm, inv_hbm, widx_hbm, rstart_hbm,
                  ids_v, hist_v, perm_v, inv_v, starts_v, ncum_v, wi_v, rs_v)


def _logits_body(inter_ref, lw_ref, b_ref, out_ref):
    logits = jax.lax.dot_general(
        inter_ref[...].astype(jnp.bfloat16), lw_ref[...],
        (((1,), (1,)), ((), ())),
        preferred_element_type=jnp.float32)
    out_ref[...] = jax.nn.sigmoid(logits + b_ref[...])


@jax.jit
def kernel(vocab_ids, compressed, decoder_weights, linear_w, linear_b):
    perm, inv_perm, w_idx, row_start = _sc_sort(vocab_ids)
    c_sorted = jnp.zeros((BP, IN_DIM), jnp.float32).at[:BATCH].set(
        compressed[perm])

    def dw_index(i, widx, rst, j):
        return (widx[i * CPER + j], 0, 0)

    in_specs = [
        pl.BlockSpec((1, IN_DIM, INTER_DIM), functools.partial(dw_index, j=j))
        for j in range(CPER)
    ]
    in_specs.append(pl.BlockSpec((BP, IN_DIM), lambda i, widx, rst: (0, 0)))

    inter_sorted = pl.pallas_call(
        _chunk_matvec_body,
        grid_spec=pltpu.PrefetchScalarGridSpec(
            num_scalar_prefetch=2,
            grid=(CMAX // CPER,),
            in_specs=in_specs,
            out_specs=pl.BlockSpec((BP, INTER_DIM), lambda i, widx, rst: (0, 0)),
        ),
        out_shape=jax.ShapeDtypeStruct((BP, INTER_DIM), jnp.float32),
    )(w_idx, row_start, *([decoder_weights] * CPER), c_sorted)

    inter = inter_sorted[:BATCH][inv_perm]

    out = pl.pallas_call(
        _logits_body,
        grid=(BATCH // RM,),
        in_specs=[
            pl.BlockSpec((RM, INTER_DIM), lambda i: (i, 0)),
            pl.BlockSpec((VOCAB, INTER_DIM), lambda i: (0, 0)),
            pl.BlockSpec((1, VOCAB), lambda i: (0, 0)),
        ],
        out_specs=pl.BlockSpec((RM, VOCAB), lambda i: (i, 0)),
        out_shape=jax.ShapeDtypeStruct((BATCH, VOCAB), jnp.float32),
    )(inter, linear_w.astype(jnp.bfloat16), linear_b.reshape(1, VOCAB))
    return out


# speedup vs baseline: 1.6622x; 1.6622x over previous
import functools

import jax
import jax.numpy as jnp
from jax import lax
from jax.experimental import pallas as pl
from jax.experimental.pallas import tpu as pltpu
from jax.experimental.pallas import tpu_sc as plsc

BATCH = 4096
IN_DIM = 128
INTER_DIM = 128
VOCAB = 1000
CH = 8
CPER = 16
CMAX = 1408
BP = BATCH + CH
RM = 512


def _chunk_matvec_body(widx_ref, rstart_ref, *refs):
    dw_refs = refs[:CPER]
    c_ref, out_ref = refs[CPER:]
    i = pl.program_id(0)
    for j in range(CPER):
        w = dw_refs[j][0].astype(jnp.bfloat16)
        k = rstart_ref[i * CPER + j]
        rows = c_ref[pl.ds(k, CH), :].astype(jnp.bfloat16)
        out_ref[pl.ds(k, CH), :] = jnp.tanh(
            jax.lax.dot(rows, w, preferred_element_type=jnp.float32))


L = 16
VP = 1024


def _sc_sort_body(ids_hbm, perm_hbm, inv_hbm, widx_hbm, rstart_hbm,
                  ids_v, hist_v, perm_v, inv_v, starts_v, ncum_v,
                  wi_v, rs_v):
    wid = lax.axis_index("s") * 2 + lax.axis_index("c")

    @pl.when(wid == 0)
    def _():
        pltpu.sync_copy(ids_hbm, ids_v)
        iota = lax.iota(jnp.int32, L)
        zeros = jnp.zeros((L,), jnp.int32)
        ones = jnp.ones((L,), jnp.int32)

        def zero_body(k, c):
            hist_v[pl.ds(k * L, L)] = zeros
            return c

        lax.fori_loop(0, (L * VP) // L, zero_body, 0)

        def hist_body(c, carry):
            vec = ids_v[pl.ds(c * L, L)]
            plsc.addupdate_scatter(hist_v, [vec + iota * VP], ones)
            return carry

        lax.fori_loop(0, BATCH // L, hist_body, 0)

        def mark_init(c, carry):
            wi_v[pl.ds(c * L, L)] = zeros - 1
            return carry

        lax.fori_loop(0, CMAX // L, mark_init, 0)

        def merge_body(k, carry):
            rcarry, ccarry = carry
            acc = hist_v[pl.ds(k * L, L)]
            for l in range(1, L):
                acc = acc + hist_v[pl.ds(l * VP + k * L, L)]
            incl = plsc.cumsum(acc)
            svec = incl - acc + rcarry
            starts_v[pl.ds(k * L, L)] = svec
            base = svec
            for l in range(L):
                tmp = hist_v[pl.ds(l * VP + k * L, L)]
                hist_v[pl.ds(l * VP + k * L, L)] = base
                base = base + tmp
            nc = lax.shift_right_logical(acc + (CH - 1), 3)
            ninc = plsc.cumsum(nc)
            nprev = ninc - nc + ccarry
            ncum_v[pl.ds(k * L, L)] = nprev
            plsc.store_scatter(wi_v, [nprev], k * L + iota, mask=acc > 0)
            return (rcarry + jnp.sum(acc), ccarry + jnp.sum(nc))

        lax.fori_loop(0, VP // L, merge_body, (0, 0))

        def fill_body(c, carry):
            vec = wi_v[pl.ds(c * L, L)]
            g = jnp.maximum(plsc.cummax(vec), carry)
            nprev = plsc.load_gather(ncum_v, [g])
            sg = plsc.load_gather(starts_v, [g])
            slot = c * L + iota
            rs = jnp.minimum(sg + CH * (slot - nprev), BATCH)
            wi_v[pl.ds(c * L, L)] = g
            rs_v[pl.ds(c * L, L)] = rs
            return jnp.max(g)

        lax.fori_loop(0, CMAX // L, fill_body, 0)

        def place_body(c, carry):
            vec = ids_v[pl.ds(c * L, L)]
            idx = vec + iota * VP
            cur = plsc.load_gather(hist_v, [idx])
            rows = c * L + iota
            plsc.store_scatter(perm_v, [cur], rows)
            inv_v[pl.ds(c * L, L)] = cur
            plsc.store_scatter(hist_v, [idx], cur + 1)
            return carry

        lax.fori_loop(0, BATCH // L, place_body, 0)

        pltpu.sync_copy(perm_v, perm_hbm)
        pltpu.sync_copy(inv_v, inv_hbm)
        pltpu.sync_copy(wi_v, widx_hbm)
        pltpu.sync_copy(rs_v, rstart_hbm)


@functools.partial(
    pl.kernel,
    out_type=[
        jax.ShapeDtypeStruct((BATCH,), jnp.int32),
        jax.ShapeDtypeStruct((BATCH,), jnp.int32),
        jax.ShapeDtypeStruct((CMAX,), jnp.int32),
        jax.ShapeDtypeStruct((CMAX,), jnp.int32),
    ],
    mesh=plsc.VectorSubcoreMesh(core_axis_name="c", subcore_axis_name="s"),
    compiler_params=pltpu.CompilerParams(needs_layout_passes=False),
    scratch_types=[
        pltpu.VMEM((BATCH,), jnp.int32),
        pltpu.VMEM((L * VP,), jnp.int32),
        pltpu.VMEM((BATCH,), jnp.int32),
        pltpu.VMEM((BATCH,), jnp.int32),
        pltpu.VMEM((VP,), jnp.int32),
        pltpu.VMEM((VP,), jnp.int32),
        pltpu.VMEM((CMAX,), jnp.int32),
        pltpu.VMEM((CMAX,), jnp.int32),
    ],
)
def _sc_sort(ids_hbm, perm_hbm, inv_hbm, widx_hbm, rstart_hbm,
             ids_v, hist_v, perm_v, inv_v, starts_v, ncum_v, wi_v, rs_v):
    _sc_sort_body(ids_hbm, perm_hbm, inv_hbm, widx_hbm, rstart_hbm,
                  ids_v, hist_v, perm_v, inv_v, starts_v, ncum_v, wi_v, rs_v)


def _logits_body(inter_ref, lw_ref, b_ref, out_ref):
    logits = jax.lax.dot_general(
        inter_ref[...].astype(jnp.bfloat16), lw_ref[...],
        (((1,), (1,)), ((), ())),
        preferred_element_type=jnp.float32)
    out_ref[...] = jax.nn.sigmoid(logits + b_ref[...])


@jax.jit
def kernel(vocab_ids, compressed, decoder_weights, linear_w, linear_b):
    perm, inv_perm, w_idx, row_start = _sc_sort(vocab_ids)
    c_sorted = jnp.zeros((BP, IN_DIM), jnp.float32).at[:BATCH].set(
        compressed[perm])

    def dw_index(i, widx, rst, j):
        return (widx[i * CPER + j], 0, 0)

    in_specs = [
        pl.BlockSpec((1, IN_DIM, INTER_DIM), functools.partial(dw_index, j=j))
        for j in range(CPER)
    ]
    in_specs.append(pl.BlockSpec((BP, IN_DIM), lambda i, widx, rst: (0, 0)))

    inter_sorted = pl.pallas_call(
        _chunk_matvec_body,
        grid_spec=pltpu.PrefetchScalarGridSpec(
            num_scalar_prefetch=2,
            grid=(CMAX // CPER,),
            in_specs=in_specs,
            out_specs=pl.BlockSpec((BP, INTER_DIM), lambda i, widx, rst: (0, 0)),
        ),
        out_shape=jax.ShapeDtypeStruct((BP, INTER_DIM), jnp.float32),
    )(w_idx, row_start, *([decoder_weights] * CPER), c_sorted)

    inter = inter_sorted[:BATCH][inv_perm]

    out = pl.pallas_call(
        _logits_body,
        grid=(BATCH // RM,),
        in_specs=[
            pl.BlockSpec((RM, INTER_DIM), lambda i: (i, 0)),
            pl.BlockSpec((VOCAB, INTER_DIM), lambda i: (0, 0)),
            pl.BlockSpec((1, VOCAB), lambda i: (0, 0)),
        ],
        out_specs=pl.BlockSpec((RM, VOCAB), lambda i: (i, 0)),
        out_shape=jax.ShapeDtypeStruct((BATCH, VOCAB), jnp.float32),
    )(inter, linear_w.astype(jnp.bfloat16), linear_b.reshape(1, VOCAB))
    return out
